# baseline (device time: 31595 ns/iter reference)
import os

import jax
import jax.numpy as jnp
from jax import lax
from jax.experimental import pallas as pl
from jax.experimental.pallas import tpu as pltpu

_KMODE = os.environ.get("KMODE", "full")

_NCHUNK = 8
_COMM_DTYPE = jnp.float8_e4m3fn


def kernel(Q, K, V):
    b, s, h, d = Q.shape
    scale = d ** -0.5
    comm = _KMODE != "nocomm"
    hq = h // 4

    Qt = jnp.transpose(Q, (0, 2, 3, 1))
    Kt = jnp.transpose(K, (0, 2, 3, 1))
    Vt = jnp.transpose(V, (0, 2, 3, 1))

    def body(q_ref, k_ref, v_ref, o_ref, kbuf, vbuf, sx, rx, sf, rf,
             exit_sem):
        my_x = lax.axis_index("x")
        my_y = lax.axis_index("y")
        my_z = lax.axis_index("z")
        xnbr = (1 - my_x, my_y, my_z)
        ynbr = (my_x, 1 - my_y, my_z)

        if comm:
            barrier_sem = pltpu.get_barrier_semaphore()
            for dev in (xnbr, ynbr):
                pl.semaphore_signal(
                    barrier_sem, inc=1, device_id=dev,
                    device_id_type=pl.DeviceIdType.MESH,
                )

        for bi in range(b):
            kbuf[0, bi] = k_ref[bi].astype(jnp.bfloat16)
            vbuf[0, bi] = v_ref[bi].astype(_COMM_DTYPE)

        def chunk_at(slot, piece, c):
            buf, hc = (kbuf, c // 2) if c % 2 == 0 else (vbuf, c // 2)
            return buf.at[slot, piece, hc * hq:(hc + 1) * hq]

        rdma_x = []
        rdma_f = []
        if comm:
            pl.semaphore_wait(barrier_sem, 2)
            for c in range(_NCHUNK):
                r = pltpu.make_async_remote_copy(
                    src_ref=chunk_at(0, my_y, c),
                    dst_ref=chunk_at(1, my_y, c),
                    send_sem=sx.at[c], recv_sem=rx.at[c],
                    device_id=xnbr, device_id_type=pl.DeviceIdType.MESH,
                )
                r.start()
                rdma_x.append(r)

        dn_tn = (((0,), (0,)), ((), ()))
        dn_nn = (((1,), (0,)), ((), ()))

        def attn_piece(kt, vt, qt):
            st = lax.dot_general(kt, qt, dn_tn,
                                 preferred_element_type=jnp.float32)
            pt = jnp.exp(st)
            l = jnp.sum(pt, axis=0, keepdims=True)
            acc = lax.dot_general(vt, pt.astype(jnp.bfloat16), dn_nn,
                                  preferred_element_type=jnp.float32)
            return acc, l

        units = [(bi, hi) for bi in range(b) for hi in range(h)]
        per_chunk = len(units) // _NCHUNK

        qts = {}
        partial = {}

        def do_local(unit_idx):
            bi, hi = units[unit_idx]
            qt = (q_ref[bi, hi] * scale).astype(jnp.bfloat16)
            qts[(bi, hi)] = qt
            partial[(bi, hi)] = attn_piece(
                k_ref[bi, hi].astype(jnp.bfloat16),
                v_ref[bi, hi].astype(jnp.bfloat16),
                qt,
            )

        ui = 0
        if comm:
            for c in range(_NCHUNK):
                for _ in range(per_chunk):
                    do_local(ui)
                    ui += 1
                rdma_x[c].wait_recv()
                r = pltpu.make_async_remote_copy(
                    src_ref=chunk_at(1, my_y, c),
                    dst_ref=chunk_at(1, my_y, c),
                    send_sem=sf.at[c], recv_sem=rf.at[c],
                    device_id=ynbr, device_id_type=pl.DeviceIdType.MESH,
                )
                r.start()
                rdma_f.append(r)
        while ui < len(units):
            do_local(ui)
            ui += 1

        if comm:
            for c in range(_NCHUNK):
                rdma_x[c].wait_send()
                rdma_f[c].wait()

        if _KMODE == "nocompute":
            o_ref[0, 0] = kbuf[1, 0, 0].astype(jnp.float32)
        else:
            rslot = 1 if comm else 0
            for bi in range(b):
                for hi in range(h):
                    acc0, l0 = partial[(bi, hi)]
                    acc1, l1 = attn_piece(
                        kbuf[rslot, bi, hi],
                        vbuf[rslot, bi, hi].astype(jnp.bfloat16),
                        qts[(bi, hi)],
                    )
                    o_ref[bi, hi] = (acc0 + acc1) * (1.0 / (l0 + l1))

        if comm:
            for dev in (xnbr, ynbr):
                pl.semaphore_signal(
                    exit_sem, inc=1, device_id=dev,
                    device_id_type=pl.DeviceIdType.MESH,
                )
            pl.semaphore_wait(exit_sem, 2)

    out_t = pl.pallas_call(
        body,
        out_shape=jax.ShapeDtypeStruct((b, h, d, s), jnp.float32),
        in_specs=[pl.BlockSpec(memory_space=pltpu.VMEM)] * 3,
        out_specs=pl.BlockSpec(memory_space=pltpu.VMEM),
        scratch_shapes=[
            pltpu.VMEM((2, b, h, d, s), jnp.bfloat16),
            pltpu.VMEM((2, b, h, d, s), _COMM_DTYPE),
            pltpu.SemaphoreType.DMA((_NCHUNK,)),
            pltpu.SemaphoreType.DMA((_NCHUNK,)),
            pltpu.SemaphoreType.DMA((_NCHUNK,)),
            pltpu.SemaphoreType.DMA((_NCHUNK,)),
            pltpu.SemaphoreType.REGULAR,
        ],
        compiler_params=pltpu.CompilerParams(
            vmem_limit_bytes=100 * 1024 * 1024,
            **({} if not comm else {"collective_id": 0}),
        ),
    )(Qt, Kt, Vt)
    return jnp.transpose(out_t, (0, 3, 1, 2))


# device time: 14619 ns/iter; 2.1612x vs baseline; 2.1612x over previous
import os

import jax
import jax.numpy as jnp
from jax import lax
from jax.experimental import pallas as pl
from jax.experimental.pallas import tpu as pltpu

_KMODE = os.environ.get("KMODE", "full")

_NDATA = 8
_NCHUNK = _NDATA + 1


def kernel(Q, K, V):
    b, s, h, d = Q.shape
    scale = d ** -0.5
    comm = _KMODE != "nocomm"
    hq = h // (_NDATA // 2)

    Qt = jnp.transpose(Q, (0, 2, 3, 1))
    Kt = jnp.transpose(K, (0, 2, 3, 1))
    Vt = jnp.transpose(V, (0, 2, 3, 1))

    def body(q_ref, k_ref, v_ref, o_ref, kbuf, vbuf, sbuf, sx, rx, sf, rf,
             exit_sem):
        my_x = lax.axis_index("x")
        my_y = lax.axis_index("y")
        my_z = lax.axis_index("z")
        xnbr = (1 - my_x, my_y, my_z)
        ynbr = (my_x, 1 - my_y, my_z)

        if comm:
            barrier_sem = pltpu.get_barrier_semaphore()
            for dev in (xnbr, ynbr):
                pl.semaphore_signal(
                    barrier_sem, inc=1, device_id=dev,
                    device_id_type=pl.DeviceIdType.MESH,
                )

        for bi in range(b):
            for ti, t_ref, tbuf in ((0, k_ref, kbuf), (1, v_ref, vbuf)):
                tf = t_ref[bi]
                amax = jnp.max(jnp.abs(tf), axis=2, keepdims=True)
                sbuf[0, bi, ti] = amax * (1.0 / 127.0)
                tbuf[0, bi] = jnp.round(tf * (127.0 / amax)).astype(jnp.int8)

        def chunk_at(slot, piece, c):
            if c == 0:
                return sbuf.at[slot, piece]
            cc = c - 1
            buf, hc = (kbuf, cc // 2) if cc % 2 == 0 else (vbuf, cc // 2)
            return buf.at[slot, piece, hc * hq:(hc + 1) * hq]

        rdma_x = []
        rdma_f = []
        if comm:
            pl.semaphore_wait(barrier_sem, 2)
            for c in range(_NCHUNK):
                r = pltpu.make_async_remote_copy(
                    src_ref=chunk_at(0, my_y, c),
                    dst_ref=chunk_at(1, my_y, c),
                    send_sem=sx.at[c], recv_sem=rx.at[c],
                    device_id=xnbr, device_id_type=pl.DeviceIdType.MESH,
                )
                r.start()
                rdma_x.append(r)

        dn_tn = (((0,), (0,)), ((), ()))
        dn_nn = (((1,), (0,)), ((), ()))

        def attn_piece(kt, vt, qt, sv=None):
            st = lax.dot_general(kt, qt, dn_tn,
                                 preferred_element_type=jnp.float32)
            pt = jnp.exp(st)
            l = jnp.sum(pt, axis=0, keepdims=True)
            acc = lax.dot_general(vt, pt.astype(jnp.bfloat16), dn_nn,
                                  preferred_element_type=jnp.float32)
            if sv is not None:
                acc = acc * sv
            return acc, l

        units = [(bi, hi) for bi in range(b) for hi in range(h)]
        per_chunk = -(-len(units) // _NCHUNK)

        qtf = {}
        qts = {}
        partial = {}

        def do_local(unit_idx):
            bi, hi = units[unit_idx]
            qf = q_ref[bi, hi] * scale
            qtf[(bi, hi)] = qf
            qt = qf.astype(jnp.bfloat16)
            qts[(bi, hi)] = qt
            partial[(bi, hi)] = attn_piece(
                k_ref[bi, hi].astype(jnp.bfloat16),
                v_ref[bi, hi].astype(jnp.bfloat16),
                qt,
            )

        ui = 0
        if comm:
            for c in range(_NCHUNK):
                rdma_x[c].wait_recv()
                r = pltpu.make_async_remote_copy(
                    src_ref=chunk_at(1, my_y, c),
                    dst_ref=chunk_at(1, my_y, c),
                    send_sem=sf.at[c], recv_sem=rf.at[c],
                    device_id=ynbr, device_id_type=pl.DeviceIdType.MESH,
                )
                r.start()
                rdma_f.append(r)
                for _ in range(per_chunk):
                    if ui < len(units):
                        do_local(ui)
                        ui += 1
        while ui < len(units):
            do_local(ui)
            ui += 1

        if comm:
            for c in range(_NCHUNK):
                rdma_x[c].wait_send()
                rdma_f[c].wait()

        if _KMODE == "nocompute":
            o_ref[0, 0] = kbuf[1, 0, 0].astype(jnp.float32)
        else:
            rslot = 1 if comm else 0
            for bi in range(b):
                for hi in range(h):
                    acc0, l0 = partial[(bi, hi)]
                    sk = sbuf[rslot, bi, 0, hi]
                    sv = sbuf[rslot, bi, 1, hi]
                    qt_r = (qtf[(bi, hi)] * sk).astype(jnp.bfloat16)
                    acc1, l1 = attn_piece(
                        kbuf[rslot, bi, hi].astype(jnp.bfloat16),
                        vbuf[rslot, bi, hi].astype(jnp.bfloat16),
                        qt_r, sv=sv,
                    )
                    o_ref[bi, hi] = (acc0 + acc1) * (1.0 / (l0 + l1))

        if comm:
            for dev in (xnbr, ynbr):
                pl.semaphore_signal(
                    exit_sem, inc=1, device_id=dev,
                    device_id_type=pl.DeviceIdType.MESH,
                )
            pl.semaphore_wait(exit_sem, 2)

    out_t = pl.pallas_call(
        body,
        out_shape=jax.ShapeDtypeStruct((b, h, d, s), jnp.float32),
        in_specs=[pl.BlockSpec(memory_space=pltpu.VMEM)] * 3,
        out_specs=pl.BlockSpec(memory_space=pltpu.VMEM),
        scratch_shapes=[
            pltpu.VMEM((2, b, h, d, s), jnp.int8),
            pltpu.VMEM((2, b, h, d, s), jnp.int8),
            pltpu.VMEM((2, b, 2, h, d, 1), jnp.float32),
            pltpu.SemaphoreType.DMA((_NCHUNK,)),
            pltpu.SemaphoreType.DMA((_NCHUNK,)),
            pltpu.SemaphoreType.DMA((_NCHUNK,)),
            pltpu.SemaphoreType.DMA((_NCHUNK,)),
            pltpu.SemaphoreType.REGULAR,
        ],
        compiler_params=pltpu.CompilerParams(
            vmem_limit_bytes=100 * 1024 * 1024,
            **({} if not comm else {"collective_id": 0}),
        ),
    )(Qt, Kt, Vt)
    return jnp.transpose(out_t, (0, 3, 1, 2))
